# Initial kernel scaffold; baseline (speedup 1.0000x reference)
import jax
import jax.numpy as jnp
from jax.experimental import pallas as pl


def kernel(inputs, table):
    raise NotImplementedError("write your pallas kernel here")


# speedup vs baseline: 1.5842x; 1.5842x over previous
import functools

import jax
import jax.numpy as jnp
from jax import lax
from jax.experimental import pallas as pl
from jax.experimental.pallas import tpu as pltpu
from jax.experimental.pallas import tpu_sc as plsc

_VOCAB = 1000000
_D = 64
_B = 4096
_NSENT = 5
_LSEQ = 50

_NC, _NSUB = 2, 16
_NW = _NC * _NSUB
_SEGS = _B * _NSENT
_ROWS = _SEGS * _LSEQ
_SEGS_PER_W = _SEGS // _NW
_ROWS_PER_W = _ROWS // _NW
_CHUNK_SEGS = 8
_CHUNK_ROWS = _CHUNK_SEGS * _LSEQ
_N_CHUNKS = _SEGS_PER_W // _CHUNK_SEGS


def _sc_segment_sums(table, flat_idx):
    mesh = plsc.VectorSubcoreMesh(core_axis_name="c", subcore_axis_name="s")

    @functools.partial(
        pl.kernel,
        out_type=jax.ShapeDtypeStruct((_SEGS, _D), jnp.float32),
        mesh=mesh,
        scratch_types=[
            pltpu.VMEM((_CHUNK_ROWS,), jnp.int32),
            pltpu.VMEM((_CHUNK_ROWS, _D), jnp.float32),
            pltpu.VMEM((_CHUNK_SEGS, _D), jnp.float32),
            pltpu.SemaphoreType.DMA,
        ],
        compiler_params=pltpu.CompilerParams(use_tc_tiling_on_sc=False),
    )
    def sums_kernel(table_hbm, idx_hbm, out_hbm, idx_v, rows_v, sums_v, sem):
        wid = lax.axis_index("s") * _NC + lax.axis_index("c")

        def chunk_body(c, carry):
            row_base = wid * _ROWS_PER_W + c * _CHUNK_ROWS
            seg_base = wid * _SEGS_PER_W + c * _CHUNK_SEGS
            pltpu.sync_copy(idx_hbm.at[pl.ds(row_base, _CHUNK_ROWS)], idx_v)
            pltpu.async_copy(table_hbm.at[idx_v], rows_v, sem).wait()

            def seg_body(s, carry2):
                def row_body(r, accs):
                    b = s * _LSEQ + r
                    return tuple(
                        accs[k] + rows_v[b, pl.ds(16 * k, 16)] for k in range(4)
                    )

                z = jnp.zeros((16,), jnp.float32)
                a = lax.fori_loop(0, _LSEQ, row_body, (z, z, z, z))
                for k in range(4):
                    sums_v[s, pl.ds(16 * k, 16)] = a[k]
                return carry2

            lax.fori_loop(0, _CHUNK_SEGS, seg_body, 0)
            pltpu.sync_copy(sums_v, out_hbm.at[pl.ds(seg_base, _CHUNK_SEGS)])
            return carry

        lax.fori_loop(0, _N_CHUNKS, chunk_body, 0)

    return sums_kernel(table, flat_idx)


def _tc_finish(sums):
    def body(x_ref, cos_ref, pred_ref):
        x = x_ref[...] * (1.0 / _LSEQ)
        anchor = x[0:_B]
        a2 = jnp.sum(anchor * anchor, axis=-1, keepdims=True)
        nums = []
        o2s = []
        for k in range(1, _NSENT):
            o = x[k * _B : (k + 1) * _B]
            nums.append(jnp.sum(anchor * o, axis=-1, keepdims=True))
            o2s.append(jnp.sum(o * o, axis=-1, keepdims=True))
        num = jnp.concatenate(nums, axis=1)
        on = jnp.sqrt(jnp.concatenate(o2s, axis=1))
        an = jnp.sqrt(a2)
        cos = num / (an * on + 1e-8)
        cos_ref[...] = cos
        m = jnp.max(cos, axis=1, keepdims=True)
        e = jnp.exp(cos - m)
        pred_ref[...] = e / jnp.sum(e, axis=1, keepdims=True)

    return pl.pallas_call(
        body,
        out_shape=(
            jax.ShapeDtypeStruct((_B, _NSENT - 1), jnp.float32),
            jax.ShapeDtypeStruct((_B, _NSENT - 1), jnp.float32),
        ),
    )(sums)


def kernel(inputs, table):
    flat_idx = inputs.transpose(1, 0, 2).reshape(-1)
    sums = _sc_segment_sums(table, flat_idx)
    return _tc_finish(sums)


# speedup vs baseline: 1.9549x; 1.2340x over previous
import functools

import jax
import jax.numpy as jnp
from jax import lax
from jax.experimental import pallas as pl
from jax.experimental.pallas import tpu as pltpu
from jax.experimental.pallas import tpu_sc as plsc

_VOCAB = 1000000
_D = 64
_B = 4096
_NSENT = 5
_LSEQ = 50

_NC, _NSUB = 2, 16
_NW = _NC * _NSUB
_SEGS = _B * _NSENT
_ROWS = _SEGS * _LSEQ
_SEGS_PER_W = _SEGS // _NW
_ROWS_PER_W = _ROWS // _NW
_CHUNK_SEGS = 8
_CHUNK_ROWS = _CHUNK_SEGS * _LSEQ
_N_CHUNKS = _SEGS_PER_W // _CHUNK_SEGS


def _sc_segment_sums(table, flat_idx):
    mesh = plsc.VectorSubcoreMesh(core_axis_name="c", subcore_axis_name="s")

    @functools.partial(
        pl.kernel,
        out_type=jax.ShapeDtypeStruct((_SEGS, _D), jnp.float32),
        mesh=mesh,
        scratch_types=[
            pltpu.VMEM((_ROWS_PER_W,), jnp.int32),
            pltpu.VMEM((_CHUNK_ROWS, _D), jnp.float32),
            pltpu.VMEM((_CHUNK_ROWS, _D), jnp.float32),
            pltpu.VMEM((_SEGS_PER_W, _D), jnp.float32),
            pltpu.SemaphoreType.DMA,
            pltpu.SemaphoreType.DMA,
        ],
        compiler_params=pltpu.CompilerParams(use_tc_tiling_on_sc=False),
    )
    def sums_kernel(table_hbm, idx_hbm, out_hbm, idx_v, rows0, rows1, acc_v,
                    sem0, sem1):
        wid = lax.axis_index("s") * _NC + lax.axis_index("c")
        pltpu.sync_copy(idx_hbm.at[pl.ds(wid * _ROWS_PER_W, _ROWS_PER_W)], idx_v)

        def start_gather(c, rows, sem):
            return pltpu.async_copy(
                table_hbm.at[idx_v.at[pl.ds(c * _CHUNK_ROWS, _CHUNK_ROWS)]],
                rows, sem,
            )

        def wait_slot0():
            pltpu.make_async_copy(
                table_hbm.at[pl.ds(0, _CHUNK_ROWS)], rows0, sem0
            ).wait()

        def accumulate(rows, c):
            def seg_body(s, carry2):
                def row_body(r, accs):
                    b = s * _LSEQ + r
                    return tuple(
                        accs[k] + rows[b, pl.ds(16 * k, 16)] for k in range(4)
                    )

                z = jnp.zeros((16,), jnp.float32)
                a = lax.fori_loop(0, _LSEQ, row_body, (z, z, z, z), unroll=10)
                for k in range(4):
                    acc_v[c * _CHUNK_SEGS + s, pl.ds(16 * k, 16)] = a[k]
                return carry2

            lax.fori_loop(0, _CHUNK_SEGS, seg_body, 0)

        start_gather(0, rows0, sem0)

        def pair_body(c2, carry):
            c0 = 2 * c2
            d1 = start_gather(c0 + 1, rows1, sem1)
            wait_slot0()
            accumulate(rows0, c0)

            @pl.when(c2 < _N_CHUNKS // 2 - 1)
            def _():
                start_gather(c0 + 2, rows0, sem0)

            d1.wait()
            accumulate(rows1, c0 + 1)
            return carry

        lax.fori_loop(0, _N_CHUNKS // 2, pair_body, 0)
        pltpu.sync_copy(acc_v, out_hbm.at[pl.ds(wid * _SEGS_PER_W, _SEGS_PER_W)])

    return sums_kernel(table, flat_idx)


def _tc_finish(sums):
    def body(x_ref, cos_ref, pred_ref):
        x = x_ref[...] * (1.0 / _LSEQ)
        anchor = x[0:_B]
        a2 = jnp.sum(anchor * anchor, axis=-1, keepdims=True)
        nums = []
        o2s = []
        for k in range(1, _NSENT):
            o = x[k * _B : (k + 1) * _B]
            nums.append(jnp.sum(anchor * o, axis=-1, keepdims=True))
            o2s.append(jnp.sum(o * o, axis=-1, keepdims=True))
        num = jnp.concatenate(nums, axis=1)
        on = jnp.sqrt(jnp.concatenate(o2s, axis=1))
        an = jnp.sqrt(a2)
        cos = num / (an * on + 1e-8)
        cos_ref[...] = cos
        m = jnp.max(cos, axis=1, keepdims=True)
        e = jnp.exp(cos - m)
        pred_ref[...] = e / jnp.sum(e, axis=1, keepdims=True)

    return pl.pallas_call(
        body,
        out_shape=(
            jax.ShapeDtypeStruct((_B, _NSENT - 1), jnp.float32),
            jax.ShapeDtypeStruct((_B, _NSENT - 1), jnp.float32),
        ),
    )(sums)


def kernel(inputs, table):
    flat_idx = inputs.transpose(1, 0, 2).reshape(-1)
    sums = _sc_segment_sums(table, flat_idx)
    return _tc_finish(sums)
